# baseline (speedup 1.0000x reference)
import jax
import jax.numpy as jnp
from jax import lax
from jax.experimental import pallas as pl
from jax.experimental.pallas import tpu as pltpu
from jax.experimental.pallas import tpu_sc as plsc

NUM_CORES = 2
NUM_SUBCORES = 16
LANES = 16
NW = NUM_CORES * NUM_SUBCORES

BATCH = 16384
EMBED_DIM = 32
BPW = BATCH // NW
CHUNK = 128
NCHUNKS = BPW // CHUNK
CGROUPS = CHUNK // LANES


def _cf_body(uidx_hbm, iidx_hbm, uemb_hbm, iemb_hbm, ubias_hbm, ibias_hbm,
             out_hbm, uidx_v, iidx_v, urows_v, irows_v, ubias_v, ibias_v,
             out_v, sem_u, sem_i, sem_ub, sem_ib):
    wid = lax.axis_index("c") * NUM_SUBCORES + lax.axis_index("s")
    base = wid * BPW

    pltpu.sync_copy(uidx_hbm.at[pl.ds(base, BPW)], uidx_v)
    pltpu.sync_copy(iidx_hbm.at[pl.ds(base, BPW)], iidx_v)

    lanes = lax.iota(jnp.int32, LANES)

    def chunk_body(c, carry):
        coff = c * CHUNK

        def issue_body(b, carry2):
            uvec = uidx_v[pl.ds(coff + b * LANES, LANES)]
            tvec = iidx_v[pl.ds(coff + b * LANES, LANES)]
            for lane in range(LANES):
                j = b * LANES + lane
                u = uvec[lane]
                t = tvec[lane]
                pltpu.async_copy(uemb_hbm.at[pl.ds(u, 1), :],
                                 urows_v.at[pl.ds(j, 1), :], sem_u)
                pltpu.async_copy(iemb_hbm.at[pl.ds(t, 1), :],
                                 irows_v.at[pl.ds(j, 1), :], sem_i)
                pltpu.async_copy(ubias_hbm.at[pl.ds(u, 1), :],
                                 ubias_v.at[pl.ds(j, 1), :], sem_ub)
                pltpu.async_copy(ibias_hbm.at[pl.ds(t, 1), :],
                                 ibias_v.at[pl.ds(j, 1), :], sem_ib)
            return carry2

        lax.fori_loop(0, CGROUPS, issue_body, 0)

        def drain_body(j, carry2):
            pltpu.make_async_copy(uemb_hbm.at[pl.ds(0, 1), :],
                                  urows_v.at[pl.ds(j, 1), :], sem_u).wait()
            pltpu.make_async_copy(iemb_hbm.at[pl.ds(0, 1), :],
                                  irows_v.at[pl.ds(j, 1), :], sem_i).wait()
            pltpu.make_async_copy(ubias_hbm.at[pl.ds(0, 1), :],
                                  ubias_v.at[pl.ds(j, 1), :], sem_ub).wait()
            pltpu.make_async_copy(ibias_hbm.at[pl.ds(0, 1), :],
                                  ibias_v.at[pl.ds(j, 1), :], sem_ib).wait()
            return carry2

        lax.fori_loop(0, CHUNK, drain_body, 0)

        def group_body(g, carry2):
            rows = lanes + g * LANES
            zero = jnp.zeros((LANES,), jnp.int32)
            acc = (plsc.load_gather(ubias_v, [rows, zero])
                   + plsc.load_gather(ibias_v, [rows, zero]))
            for d in range(EMBED_DIM):
                col = jnp.full((LANES,), d, jnp.int32)
                u = plsc.load_gather(urows_v, [rows, col])
                v = plsc.load_gather(irows_v, [rows, col])
                acc = acc + u * v
            out_v[pl.ds(coff + g * LANES, LANES)] = acc
            return carry2

        lax.fori_loop(0, CGROUPS, group_body, 0)
        return carry

    lax.fori_loop(0, NCHUNKS, chunk_body, 0)

    pltpu.sync_copy(out_v, out_hbm.at[pl.ds(base, BPW)])


_cf_kernel = pl.kernel(
    _cf_body,
    out_type=jax.ShapeDtypeStruct((BATCH,), jnp.float32),
    mesh=plsc.VectorSubcoreMesh(core_axis_name="c", subcore_axis_name="s"),
    compiler_params=pltpu.CompilerParams(needs_layout_passes=False),
    scratch_types=[
        pltpu.VMEM((BPW,), jnp.int32),
        pltpu.VMEM((BPW,), jnp.int32),
        pltpu.VMEM((CHUNK, EMBED_DIM), jnp.float32),
        pltpu.VMEM((CHUNK, EMBED_DIM), jnp.float32),
        pltpu.VMEM((CHUNK, 1), jnp.float32),
        pltpu.VMEM((CHUNK, 1), jnp.float32),
        pltpu.VMEM((BPW,), jnp.float32),
        pltpu.SemaphoreType.DMA,
        pltpu.SemaphoreType.DMA,
        pltpu.SemaphoreType.DMA,
        pltpu.SemaphoreType.DMA,
    ],
)


@jax.jit
def kernel(user_indices, item_indices, user_emb_table, item_emb_table,
           user_bias_table, item_bias_table):
    return _cf_kernel(user_indices, item_indices, user_emb_table,
                      item_emb_table, user_bias_table, item_bias_table)
